# Initial kernel scaffold; baseline (speedup 1.0000x reference)
import jax
import jax.numpy as jnp
from jax.experimental import pallas as pl


def kernel(query, key, value, Wq, bq, Wk, bk, Wv, bv, Wo, bo, centroids):
    raise NotImplementedError("write your pallas kernel here")


# speedup vs baseline: 10.9093x; 10.9093x over previous
import functools

import jax
import jax.numpy as jnp
from jax.experimental import pallas as pl

H = 16
C = 32
TOPK = 8


def _mm_kernel(x_ref, w_ref, b_ref, o_ref):
    o_ref[0] = (
        jnp.dot(x_ref[0], w_ref[0], preferred_element_type=jnp.float32)
        + b_ref[0]
    )


def _stacked_matmul(x, w, b, bm, interpret=False):
    G, M, D1 = x.shape
    D2 = w.shape[2]
    return pl.pallas_call(
        _mm_kernel,
        grid=(G, M // bm),
        in_specs=[
            pl.BlockSpec((1, bm, D1), lambda i, j: (i, j, 0)),
            pl.BlockSpec((1, D1, D2), lambda i, j: (i, 0, 0)),
            pl.BlockSpec((1, 1, D2), lambda i, j: (i, 0, 0)),
        ],
        out_specs=pl.BlockSpec((1, bm, D2), lambda i, j: (i, j, 0)),
        out_shape=jax.ShapeDtypeStruct((G, M, D2), jnp.float32),
        interpret=interpret,
    )(x, w, b)


def _first_max_mask(x):
    m = jnp.max(x, axis=-1, keepdims=True)
    eq = x == m
    ii = jax.lax.broadcasted_iota(jnp.int32, x.shape, len(x.shape) - 1)
    big = jnp.where(eq, ii, x.shape[-1])
    amin = jnp.min(big, axis=-1, keepdims=True)
    return jnp.logical_and(eq, ii == amin)


def _cluster_attn_kernel(scale, q_ref, k_ref, v_ref, c_ref, aw_ref, o_ref):
    kh = k_ref[0, 0]
    vh = v_ref[0, 0]
    qh = q_ref[0, 0]
    cen = c_ref[0]

    kn = kh / jnp.maximum(
        jnp.sqrt(jnp.sum(kh * kh, axis=-1, keepdims=True)), 1e-12)
    cn = cen / jnp.maximum(
        jnp.sqrt(jnp.sum(cen * cen, axis=-1, keepdims=True)), 1e-12)
    sims = jax.lax.dot_general(
        kn, cn, (((1,), (1,)), ((), ())),
        preferred_element_type=jnp.float32)

    oh = _first_max_mask(sims).astype(jnp.float32)
    ones = jnp.ones((kh.shape[0], 1), dtype=jnp.float32)
    counts = jax.lax.dot_general(
        oh, ones, (((0,), (0,)), ((), ())),
        preferred_element_type=jnp.float32)
    ksum = jax.lax.dot_general(
        oh, kh, (((0,), (0,)), ((), ())),
        preferred_element_type=jnp.float32)
    vsum = jax.lax.dot_general(
        oh, vh, (((0,), (0,)), ((), ())),
        preferred_element_type=jnp.float32)
    denom = jnp.maximum(counts, 1.0)
    has = counts > 0.0
    kmean = jnp.where(has, ksum / denom, cen)
    vmean = jnp.where(has, vsum / denom, 0.0)

    scores = jax.lax.dot_general(
        qh, kmean, (((1,), (1,)), ((), ())),
        preferred_element_type=jnp.float32) * scale

    gmax = jnp.max(scores, axis=-1, keepdims=True)
    work = scores
    sel = jnp.zeros(scores.shape, dtype=jnp.bool_)
    for _ in range(TOPK):
        f = _first_max_mask(work)
        sel = jnp.logical_or(sel, f)
        work = jnp.where(f, -jnp.inf, work)
    e = jnp.where(sel, jnp.exp(scores - gmax), 0.0)
    aw = e / jnp.sum(e, axis=-1, keepdims=True)

    aw_ref[0, 0] = aw
    o_ref[0, 0] = jnp.dot(aw, vmean, preferred_element_type=jnp.float32)


def _cluster_attn(q4, k4, v4, cen, scale, interpret=False):
    B, H_, Qlen, hd = q4.shape
    S = k4.shape[2]
    C_ = cen.shape[1]
    return pl.pallas_call(
        functools.partial(_cluster_attn_kernel, scale),
        grid=(B, H_),
        in_specs=[
            pl.BlockSpec((1, 1, Qlen, hd), lambda b, h: (b, h, 0, 0)),
            pl.BlockSpec((1, 1, S, hd), lambda b, h: (b, h, 0, 0)),
            pl.BlockSpec((1, 1, S, hd), lambda b, h: (b, h, 0, 0)),
            pl.BlockSpec((1, C_, hd), lambda b, h: (h, 0, 0)),
        ],
        out_specs=[
            pl.BlockSpec((1, 1, Qlen, C_), lambda b, h: (b, h, 0, 0)),
            pl.BlockSpec((1, 1, Qlen, hd), lambda b, h: (b, h, 0, 0)),
        ],
        out_shape=[
            jax.ShapeDtypeStruct((B, H_, Qlen, C_), jnp.float32),
            jax.ShapeDtypeStruct((B, H_, Qlen, hd), jnp.float32),
        ],
        interpret=interpret,
    )(q4, k4, v4, cen)


def _impl(query, key, value, Wq, bq, Wk, bk, Wv, bv, Wo, bo, centroids,
          interpret=False):
    B, Qlen, D = query.shape
    S = key.shape[1]
    hd = D // H
    scale = hd ** (-0.5)

    x = jnp.stack([
        query.reshape(B * Qlen, D),
        key.reshape(B * S, D),
        value.reshape(B * S, D),
    ])
    w = jnp.stack([Wq.T, Wk.T, Wv.T])
    b = jnp.stack([bq, bk, bv])[:, None, :]
    qkv = _stacked_matmul(x, w, b, bm=512, interpret=interpret)

    q4 = qkv[0].reshape(B, Qlen, H, hd).transpose(0, 2, 1, 3)
    k4 = qkv[1].reshape(B, S, H, hd).transpose(0, 2, 1, 3)
    v4 = qkv[2].reshape(B, S, H, hd).transpose(0, 2, 1, 3)

    attn, head_out = _cluster_attn(q4, k4, v4, centroids, scale,
                                   interpret=interpret)

    y = head_out.transpose(0, 2, 1, 3).reshape(1, B * Qlen, D)
    out = _stacked_matmul(y, Wo.T[None], bo[None, None, :], bm=512,
                          interpret=interpret)
    return out[0].reshape(B, Qlen, D), attn


def kernel(query, key, value, Wq, bq, Wk, bk, Wv, bv, Wo, bo, centroids):
    return _impl(query, key, value, Wq, bq, Wk, bk, Wv, bv, Wo, bo,
                 centroids)


# speedup vs baseline: 17.6085x; 1.6141x over previous
import functools

import jax
import jax.numpy as jnp
from jax.experimental import pallas as pl
from jax.experimental.pallas import tpu as pltpu

H = 16
C = 32
TOPK = 8


def _mm_kernel(x_ref, w_ref, b_ref, o_ref):
    o_ref[0] = (
        jnp.dot(x_ref[0], w_ref[0], preferred_element_type=jnp.float32)
        + b_ref[0]
    )


def _stacked_matmul(x, w, b, bm, interpret=False):
    G, M, D1 = x.shape
    D2 = w.shape[2]
    return pl.pallas_call(
        _mm_kernel,
        grid=(G, M // bm),
        in_specs=[
            pl.BlockSpec((1, bm, D1), lambda i, j: (i, j, 0)),
            pl.BlockSpec((1, D1, D2), lambda i, j: (i, 0, 0)),
            pl.BlockSpec((1, 1, D2), lambda i, j: (i, 0, 0)),
        ],
        out_specs=pl.BlockSpec((1, bm, D2), lambda i, j: (i, j, 0)),
        out_shape=jax.ShapeDtypeStruct((G, M, D2), jnp.float32),
        compiler_params=pltpu.CompilerParams(
            dimension_semantics=("parallel", "parallel")),
        interpret=interpret,
    )(x, w, b)


def _first_max_mask_ax0(x):
    m = jnp.max(x, axis=0, keepdims=True)
    eq = x == m
    ii = jax.lax.broadcasted_iota(jnp.int32, x.shape, 0)
    big = jnp.where(eq, ii, x.shape[0])
    amin = jnp.min(big, axis=0, keepdims=True)
    return jnp.logical_and(eq, ii == amin)


def _cluster_attn_kernel(scale, q_ref, k_ref, v_ref, c_ref,
                         aw_ref, o_ref):
    kh = k_ref[0, 0]
    vh = v_ref[0, 0]
    qh = q_ref[0, 0]
    cen = c_ref[0]

    kn = kh / jnp.maximum(
        jnp.sqrt(jnp.sum(kh * kh, axis=-1, keepdims=True)), 1e-12)
    cn = cen / jnp.maximum(
        jnp.sqrt(jnp.sum(cen * cen, axis=-1, keepdims=True)), 1e-12)
    simsT = jax.lax.dot_general(
        cn, kn, (((1,), (1,)), ((), ())),
        preferred_element_type=jnp.float32)

    ohT = _first_max_mask_ax0(simsT).astype(jnp.float32)
    counts = jnp.sum(ohT, axis=1, keepdims=True)
    ksum = jax.lax.dot_general(
        ohT, kh, (((1,), (0,)), ((), ())),
        preferred_element_type=jnp.float32)
    vsum = jax.lax.dot_general(
        ohT, vh, (((1,), (0,)), ((), ())),
        preferred_element_type=jnp.float32)
    denom = jnp.maximum(counts, 1.0)
    has = counts > 0.0
    kmean = jnp.where(has, ksum / denom, cen)
    vmean = jnp.where(has, vsum / denom, 0.0)

    scoresT = jax.lax.dot_general(
        kmean, qh, (((1,), (1,)), ((), ())),
        preferred_element_type=jnp.float32) * scale

    gmax = jnp.max(scoresT, axis=0, keepdims=True)
    work = scoresT
    sel = jnp.zeros(scoresT.shape, dtype=jnp.bool_)
    for _ in range(TOPK):
        f = _first_max_mask_ax0(work)
        sel = jnp.logical_or(sel, f)
        work = jnp.where(f, -jnp.inf, work)
    e = jnp.where(sel, jnp.exp(scoresT - gmax), 0.0)
    awT = e / jnp.sum(e, axis=0, keepdims=True)

    aw_ref[0, 0] = awT
    o_ref[0, 0] = jax.lax.dot_general(
        awT, vmean, (((0,), (0,)), ((), ())),
        preferred_element_type=jnp.float32)


def _cluster_attn(q4, k4, v4, cen, scale, interpret=False):
    B, H_, S, hd = q4.shape
    C_ = cen.shape[1]
    return pl.pallas_call(
        functools.partial(_cluster_attn_kernel, scale),
        grid=(B, H_),
        in_specs=[
            pl.BlockSpec((1, 1, S, hd), lambda b, h: (b, h, 0, 0)),
            pl.BlockSpec((1, 1, S, hd), lambda b, h: (b, h, 0, 0)),
            pl.BlockSpec((1, 1, S, hd), lambda b, h: (b, h, 0, 0)),
            pl.BlockSpec((1, C_, hd), lambda b, h: (h, 0, 0)),
        ],
        out_specs=[
            pl.BlockSpec((1, 1, C_, S), lambda b, h: (b, h, 0, 0)),
            pl.BlockSpec((1, 1, S, hd), lambda b, h: (b, h, 0, 0)),
        ],
        out_shape=[
            jax.ShapeDtypeStruct((B, H, C_, S), jnp.float32),
            jax.ShapeDtypeStruct((B, H, S, hd), jnp.float32),
        ],
        compiler_params=pltpu.CompilerParams(
            dimension_semantics=("parallel", "parallel")),
        interpret=interpret,
    )(q4, k4, v4, cen)


def _impl(query, key, value, Wq, bq, Wk, bk, Wv, bv, Wo, bo, centroids,
          interpret=False):
    B, Qlen, D = query.shape
    S = key.shape[1]
    hd = D // H
    scale = hd ** (-0.5)

    x = jnp.stack([
        query.reshape(B * Qlen, D),
        key.reshape(B * S, D),
        value.reshape(B * S, D),
    ])
    w = jnp.stack([Wq.T, Wk.T, Wv.T])
    b = jnp.stack([bq, bk, bv])[:, None, :]
    qkv = _stacked_matmul(x, w, b, bm=512, interpret=interpret)

    q4 = qkv[0].reshape(B, Qlen, H, hd).transpose(0, 2, 1, 3)
    k4 = qkv[1].reshape(B, S, H, hd).transpose(0, 2, 1, 3)
    v4 = qkv[2].reshape(B, S, H, hd).transpose(0, 2, 1, 3)
    awT, head_out = _cluster_attn(q4, k4, v4, centroids, scale,
                                  interpret=interpret)
    attn = awT.transpose(0, 1, 3, 2)

    y = head_out.transpose(0, 2, 1, 3).reshape(1, B * Qlen, D)
    out = _stacked_matmul(y, Wo.T[None], bo[None, None, :], bm=512,
                          interpret=interpret)
    return out[0].reshape(B, Qlen, D), attn


def kernel(query, key, value, Wq, bq, Wk, bk, Wv, bv, Wo, bo, centroids):
    return _impl(query, key, value, Wq, bq, Wk, bk, Wv, bv, Wo, bo,
                 centroids)


# speedup vs baseline: 35.2475x; 2.0017x over previous
import functools

import jax
import jax.numpy as jnp
from jax.experimental import pallas as pl
from jax.experimental.pallas import tpu as pltpu

H = 16
C = 32
TOPK = 8


def _qkv_kernel(q_ref, k_ref, v_ref, wq_ref, wk_ref, wv_ref,
                bq_ref, bk_ref, bv_ref, o_ref):
    o_ref[0] = jnp.dot(q_ref[...], wq_ref[...],
                       preferred_element_type=jnp.float32) + bq_ref[...]
    o_ref[1] = jnp.dot(k_ref[...], wk_ref[...],
                       preferred_element_type=jnp.float32) + bk_ref[...]
    o_ref[2] = jnp.dot(v_ref[...], wv_ref[...],
                       preferred_element_type=jnp.float32) + bv_ref[...]


def _qkv_proj(x_q, x_k, x_v, wqT, wkT, wvT, bq, bk, bv, bm,
              interpret=False):
    M, D = x_q.shape
    row = pl.BlockSpec((bm, D), lambda j: (j, 0))
    full = pl.BlockSpec((D, D), lambda j: (0, 0))
    vec = pl.BlockSpec((1, D), lambda j: (0, 0))
    return pl.pallas_call(
        _qkv_kernel,
        grid=(M // bm,),
        in_specs=[row, row, row, full, full, full, vec, vec, vec],
        out_specs=pl.BlockSpec((3, bm, D), lambda j: (0, j, 0)),
        out_shape=jax.ShapeDtypeStruct((3, M, D), jnp.float32),
        compiler_params=pltpu.CompilerParams(
            dimension_semantics=("parallel",)),
        interpret=interpret,
    )(x_q, x_k, x_v, wqT, wkT, wvT,
      bq.reshape(1, D), bk.reshape(1, D), bv.reshape(1, D))


def _first_max_mask_ax0(x):
    m = jnp.max(x, axis=0, keepdims=True)
    eq = x == m
    ii = jax.lax.broadcasted_iota(jnp.int32, x.shape, 0)
    big = jnp.where(eq, ii, x.shape[0])
    amin = jnp.min(big, axis=0, keepdims=True)
    return jnp.logical_and(eq, ii == amin)


def _one_head(kh, vh, qh, cen, scale):
    kn = kh / jnp.maximum(
        jnp.sqrt(jnp.sum(kh * kh, axis=-1, keepdims=True)), 1e-12)
    cn = cen / jnp.maximum(
        jnp.sqrt(jnp.sum(cen * cen, axis=-1, keepdims=True)), 1e-12)
    simsT = jax.lax.dot_general(
        cn, kn, (((1,), (1,)), ((), ())),
        preferred_element_type=jnp.float32)

    ohT = _first_max_mask_ax0(simsT).astype(jnp.float32)
    counts = jnp.sum(ohT, axis=1, keepdims=True)
    ksum = jax.lax.dot_general(
        ohT, kh, (((1,), (0,)), ((), ())),
        preferred_element_type=jnp.float32)
    vsum = jax.lax.dot_general(
        ohT, vh, (((1,), (0,)), ((), ())),
        preferred_element_type=jnp.float32)
    denom = jnp.maximum(counts, 1.0)
    has = counts > 0.0
    kmean = jnp.where(has, ksum / denom, cen)
    vmean = jnp.where(has, vsum / denom, 0.0)

    scoresT = jax.lax.dot_general(
        kmean, qh, (((1,), (1,)), ((), ())),
        preferred_element_type=jnp.float32) * scale

    gmax = jnp.max(scoresT, axis=0, keepdims=True)
    work = scoresT
    sel = jnp.zeros(scoresT.shape, dtype=jnp.bool_)
    for _ in range(TOPK):
        f = _first_max_mask_ax0(work)
        sel = jnp.logical_or(sel, f)
        work = jnp.where(f, -jnp.inf, work)
    e = jnp.where(sel, jnp.exp(scoresT - gmax), 0.0)
    awT = e / jnp.sum(e, axis=0, keepdims=True)

    out_h = jax.lax.dot_general(
        awT, vmean, (((0,), (0,)), ((), ())),
        preferred_element_type=jnp.float32)
    return awT, out_h


def _fused_kernel(scale, hd, q_ref, k_ref, v_ref, c_ref, wo_ref, bo_ref,
                  aw_ref, o_ref):
    h = pl.program_id(1)
    qb = q_ref[0]
    kb = k_ref[0]
    vb = v_ref[0]
    parts = []
    for i in range(2):
        sl = slice(hd * i, hd * (i + 1))
        awT, out_h = _one_head(kb[:, sl], vb[:, sl], qb[:, sl],
                               c_ref[i], scale)
        aw_ref[0, i] = awT.T
        parts.append(out_h)
    y = jnp.concatenate(parts, axis=1)
    partial = jnp.dot(y, wo_ref[...],
                      preferred_element_type=jnp.float32)

    @pl.when(h == 0)
    def _():
        o_ref[0] = partial + bo_ref[...]

    @pl.when(h != 0)
    def _():
        o_ref[0] += partial


def _fused_attn(qkv, cen, woT, bo, B, S, scale, interpret=False):
    C_, hd = cen.shape[1], cen.shape[2]
    D = qkv.shape[2]
    return pl.pallas_call(
        functools.partial(_fused_kernel, scale, hd),
        grid=(B, H // 2),
        in_specs=[
            pl.BlockSpec((1, S, 2 * hd), lambda b, h: (0, b, h)),
            pl.BlockSpec((1, S, 2 * hd), lambda b, h: (1, b, h)),
            pl.BlockSpec((1, S, 2 * hd), lambda b, h: (2, b, h)),
            pl.BlockSpec((2, C_, hd), lambda b, h: (h, 0, 0)),
            pl.BlockSpec((2 * hd, D), lambda b, h: (h, 0)),
            pl.BlockSpec((1, D), lambda b, h: (0, 0)),
        ],
        out_specs=[
            pl.BlockSpec((1, 2, S, C_), lambda b, h: (b, h, 0, 0)),
            pl.BlockSpec((1, S, D), lambda b, h: (b, 0, 0)),
        ],
        out_shape=[
            jax.ShapeDtypeStruct((B, H, S, C_), jnp.float32),
            jax.ShapeDtypeStruct((B, S, D), jnp.float32),
        ],
        compiler_params=pltpu.CompilerParams(
            dimension_semantics=("parallel", "arbitrary")),
        interpret=interpret,
    )(qkv, qkv, qkv, cen, woT, bo.reshape(1, D))


def _impl(query, key, value, Wq, bq, Wk, bk, Wv, bv, Wo, bo, centroids,
          interpret=False):
    B, Qlen, D = query.shape
    S = key.shape[1]
    hd = D // H
    scale = hd ** (-0.5)

    qkv = _qkv_proj(query.reshape(B * Qlen, D), key.reshape(B * S, D),
                    value.reshape(B * S, D), Wq.T, Wk.T, Wv.T, bq, bk, bv,
                    bm=512, interpret=interpret)
    attn, out = _fused_attn(qkv, centroids, Wo.T, bo, B, S, scale,
                            interpret=interpret)
    return out, attn


def kernel(query, key, value, Wq, bq, Wk, bk, Wv, bv, Wo, bo, centroids):
    return _impl(query, key, value, Wq, bq, Wk, bk, Wv, bv, Wo, bo,
                 centroids)


# speedup vs baseline: 42.5880x; 1.2083x over previous
import functools

import jax
import jax.numpy as jnp
from jax.experimental import pallas as pl
from jax.experimental.pallas import tpu as pltpu

H = 16
C = 32
TOPK = 8


def _qkv_kernel(q_ref, k_ref, v_ref, wq_ref, wk_ref, wv_ref,
                bq_ref, bk_ref, bv_ref, o_ref):
    o_ref[0] = jnp.dot(q_ref[...], wq_ref[...],
                       preferred_element_type=jnp.float32) + bq_ref[...]
    o_ref[1] = jnp.dot(k_ref[...], wk_ref[...],
                       preferred_element_type=jnp.float32) + bk_ref[...]
    o_ref[2] = jnp.dot(v_ref[...], wv_ref[...],
                       preferred_element_type=jnp.float32) + bv_ref[...]


def _qkv_proj(x_q, x_k, x_v, wqT, wkT, wvT, bq, bk, bv, bm,
              interpret=False):
    M, D = x_q.shape
    row = pl.BlockSpec((bm, D), lambda j: (j, 0))
    full = pl.BlockSpec((D, D), lambda j: (0, 0))
    vec = pl.BlockSpec((1, D), lambda j: (0, 0))
    return pl.pallas_call(
        _qkv_kernel,
        grid=(M // bm,),
        in_specs=[row, row, row, full, full, full, vec, vec, vec],
        out_specs=pl.BlockSpec((3, bm, D), lambda j: (0, j, 0)),
        out_shape=jax.ShapeDtypeStruct((3, M, D), jnp.float32),
        compiler_params=pltpu.CompilerParams(
            dimension_semantics=("parallel",)),
        interpret=interpret,
    )(x_q, x_k, x_v, wqT, wkT, wvT,
      bq.reshape(1, D), bk.reshape(1, D), bv.reshape(1, D))


def _first_max_mask_ax0(x):
    m = jnp.max(x, axis=0, keepdims=True)
    eq = x == m
    ii = jax.lax.broadcasted_iota(jnp.int32, x.shape, 0)
    big = jnp.where(eq, ii, x.shape[0])
    amin = jnp.min(big, axis=0, keepdims=True)
    return jnp.logical_and(eq, ii == amin)


def _topk_softmax(scoresT):
    gmax = jnp.max(scoresT, axis=0, keepdims=True)
    work = scoresT
    selm = jnp.zeros(scoresT.shape, dtype=jnp.bool_)
    for _ in range(TOPK):
        f = _first_max_mask_ax0(work)
        selm = jnp.logical_or(selm, f)
        work = jnp.where(f, -jnp.inf, work)
    e = jnp.where(selm, jnp.exp(scoresT - gmax), 0.0)
    return e / jnp.sum(e, axis=0, keepdims=True)


def _block_diag2(a, b):
    z = jnp.zeros(a.shape, dtype=a.dtype)
    return jnp.concatenate(
        [jnp.concatenate([a, z], axis=1),
         jnp.concatenate([z, b], axis=1)], axis=0)


def _fused_kernel(scale, hd, q_ref, k_ref, v_ref, c_ref, wo_ref, bo_ref,
                  aw_ref, o_ref):
    h = pl.program_id(1)
    qb = q_ref[0]
    kb = k_ref[0]
    vb = v_ref[0]
    S = kb.shape[0]

    lane = jax.lax.broadcasted_iota(jnp.int32, kb.shape, 1)
    lo = lane < hd
    sq = kb * kb
    n0 = jnp.sum(jnp.where(lo, sq, 0.0), axis=1, keepdims=True)
    n1 = jnp.sum(jnp.where(lo, 0.0, sq), axis=1, keepdims=True)
    nrm = jnp.maximum(jnp.sqrt(jnp.where(lo, n0, n1)), 1e-12)
    kn = kb / nrm

    cen0 = c_ref[0]
    cen1 = c_ref[1]
    cn0 = cen0 / jnp.maximum(
        jnp.sqrt(jnp.sum(cen0 * cen0, axis=-1, keepdims=True)), 1e-12)
    cn1 = cen1 / jnp.maximum(
        jnp.sqrt(jnp.sum(cen1 * cen1, axis=-1, keepdims=True)), 1e-12)
    cn2 = _block_diag2(cn0, cn1)

    simsT = jax.lax.dot_general(
        cn2, kn, (((1,), (1,)), ((), ())),
        preferred_element_type=jnp.float32)

    oh0 = _first_max_mask_ax0(simsT[:C]).astype(jnp.float32)
    oh1 = _first_max_mask_ax0(simsT[C:]).astype(jnp.float32)
    counts0 = jnp.sum(oh0, axis=1, keepdims=True)
    counts1 = jnp.sum(oh1, axis=1, keepdims=True)
    ohT = jnp.concatenate([oh0, oh1], axis=0)
    ksum2 = jax.lax.dot_general(
        ohT, kb, (((1,), (0,)), ((), ())),
        preferred_element_type=jnp.float32)
    vsum2 = jax.lax.dot_general(
        ohT, vb, (((1,), (0,)), ((), ())),
        preferred_element_type=jnp.float32)

    ksum0 = ksum2[:C, :hd]
    ksum1 = ksum2[C:, hd:]
    vsum0 = vsum2[:C, :hd]
    vsum1 = vsum2[C:, hd:]
    has0, den0 = counts0 > 0.0, jnp.maximum(counts0, 1.0)
    has1, den1 = counts1 > 0.0, jnp.maximum(counts1, 1.0)
    kmean0 = jnp.where(has0, ksum0 / den0, cen0)
    kmean1 = jnp.where(has1, ksum1 / den1, cen1)
    vmean0 = jnp.where(has0, vsum0 / den0, 0.0)
    vmean1 = jnp.where(has1, vsum1 / den1, 0.0)

    km2 = _block_diag2(kmean0, kmean1)
    scoresT = jax.lax.dot_general(
        km2, qb, (((1,), (1,)), ((), ())),
        preferred_element_type=jnp.float32) * scale

    aw0 = _topk_softmax(scoresT[:C])
    aw1 = _topk_softmax(scoresT[C:])

    awT2 = jnp.concatenate([aw0, aw1], axis=0)
    vm2 = _block_diag2(vmean0, vmean1)
    y = jax.lax.dot_general(
        awT2, vm2, (((0,), (0,)), ((), ())),
        preferred_element_type=jnp.float32)
    partial = jnp.dot(y, wo_ref[...],
                      preferred_element_type=jnp.float32)

    aw_ref[0, 0] = aw0.T
    aw_ref[0, 1] = aw1.T

    @pl.when(h == 0)
    def _():
        o_ref[0] = partial + bo_ref[...]

    @pl.when(h != 0)
    def _():
        o_ref[0] += partial


def _fused_attn(qkv, cen, woT, bo, B, S, scale, interpret=False):
    C_, hd = cen.shape[1], cen.shape[2]
    D = qkv.shape[2]
    return pl.pallas_call(
        functools.partial(_fused_kernel, scale, hd),
        grid=(B, H // 2),
        in_specs=[
            pl.BlockSpec((1, S, 2 * hd), lambda b, h: (0, b, h)),
            pl.BlockSpec((1, S, 2 * hd), lambda b, h: (1, b, h)),
            pl.BlockSpec((1, S, 2 * hd), lambda b, h: (2, b, h)),
            pl.BlockSpec((2, C_, hd), lambda b, h: (h, 0, 0)),
            pl.BlockSpec((2 * hd, D), lambda b, h: (h, 0)),
            pl.BlockSpec((1, D), lambda b, h: (0, 0)),
        ],
        out_specs=[
            pl.BlockSpec((1, 2, S, C_), lambda b, h: (b, h, 0, 0)),
            pl.BlockSpec((1, S, D), lambda b, h: (b, 0, 0)),
        ],
        out_shape=[
            jax.ShapeDtypeStruct((B, H, S, C_), jnp.float32),
            jax.ShapeDtypeStruct((B, S, D), jnp.float32),
        ],
        compiler_params=pltpu.CompilerParams(
            dimension_semantics=("parallel", "arbitrary")),
        interpret=interpret,
    )(qkv, qkv, qkv, cen, woT, bo.reshape(1, D))


def _impl(query, key, value, Wq, bq, Wk, bk, Wv, bv, Wo, bo, centroids,
          interpret=False):
    B, Qlen, D = query.shape
    S = key.shape[1]
    hd = D // H
    scale = hd ** (-0.5)

    qkv = _qkv_proj(query.reshape(B * Qlen, D), key.reshape(B * S, D),
                    value.reshape(B * S, D), Wq.T, Wk.T, Wv.T, bq, bk, bv,
                    bm=512, interpret=interpret)
    attn, out = _fused_attn(qkv, centroids, Wo.T, bo, B, S, scale,
                            interpret=interpret)
    return out, attn


def kernel(query, key, value, Wq, bq, Wk, bk, Wv, bv, Wo, bo, centroids):
    return _impl(query, key, value, Wq, bq, Wk, bk, Wv, bv, Wo, bo,
                 centroids)


# speedup vs baseline: 46.8352x; 1.0997x over previous
import functools

import jax
import jax.numpy as jnp
from jax.experimental import pallas as pl
from jax.experimental.pallas import tpu as pltpu

H = 16
C = 32
TOPK = 8
G = 4


def _qkv_kernel(q_ref, k_ref, v_ref, wq_ref, wk_ref, wv_ref,
                bq_ref, bk_ref, bv_ref, o_ref):
    o_ref[0] = jnp.dot(q_ref[...], wq_ref[...],
                       preferred_element_type=jnp.float32) + bq_ref[...]
    o_ref[1] = jnp.dot(k_ref[...], wk_ref[...],
                       preferred_element_type=jnp.float32) + bk_ref[...]
    o_ref[2] = jnp.dot(v_ref[...], wv_ref[...],
                       preferred_element_type=jnp.float32) + bv_ref[...]


def _qkv_proj(x_q, x_k, x_v, wqT, wkT, wvT, bq, bk, bv, bm,
              interpret=False):
    M, D = x_q.shape
    row = pl.BlockSpec((bm, D), lambda j: (j, 0))
    full = pl.BlockSpec((D, D), lambda j: (0, 0))
    vec = pl.BlockSpec((1, D), lambda j: (0, 0))
    return pl.pallas_call(
        _qkv_kernel,
        grid=(M // bm,),
        in_specs=[row, row, row, full, full, full, vec, vec, vec],
        out_specs=pl.BlockSpec((3, bm, D), lambda j: (0, j, 0)),
        out_shape=jax.ShapeDtypeStruct((3, M, D), jnp.float32),
        compiler_params=pltpu.CompilerParams(
            dimension_semantics=("parallel",)),
        interpret=interpret,
    )(x_q, x_k, x_v, wqT, wkT, wvT,
      bq.reshape(1, D), bk.reshape(1, D), bv.reshape(1, D))


def _first_max_mask_ax1(x):
    m = jnp.max(x, axis=1, keepdims=True)
    eq = x == m
    ii = jax.lax.broadcasted_iota(jnp.int32, x.shape, 1)
    big = jnp.where(eq, ii, x.shape[1])
    amin = jnp.min(big, axis=1, keepdims=True)
    return jnp.logical_and(eq, ii == amin), m


def _topk_softmax3(sc3):
    work = sc3
    selm = jnp.zeros(sc3.shape, dtype=jnp.bool_)
    gmax = None
    for _ in range(TOPK):
        f, m = _first_max_mask_ax1(work)
        gmax = m if gmax is None else gmax
        selm = jnp.logical_or(selm, f)
        work = jnp.where(f, -jnp.inf, work)
    e = jnp.where(selm, jnp.exp(sc3 - gmax), 0.0)
    return e / jnp.sum(e, axis=1, keepdims=True)


def _block_diag(mats):
    z = jnp.zeros(mats[0].shape, dtype=mats[0].dtype)
    rows = []
    for i, a in enumerate(mats):
        rows.append(jnp.concatenate(
            [a if j == i else z for j in range(len(mats))], axis=1))
    return jnp.concatenate(rows, axis=0)


def _fused_kernel(scale, hd, q_ref, k_ref, v_ref, c_ref, wo_ref, bo_ref,
                  aw_ref, o_ref):
    h = pl.program_id(1)
    qb = q_ref[0]
    kb = k_ref[0]
    vb = v_ref[0]
    S = kb.shape[0]

    lane = jax.lax.broadcasted_iota(jnp.int32, kb.shape, 1)
    head_of_lane = lane // hd
    sq = kb * kb
    nrm = jnp.zeros(kb.shape, dtype=jnp.float32)
    for i in range(G):
        sel = head_of_lane == i
        ni = jnp.sum(jnp.where(sel, sq, 0.0), axis=1, keepdims=True)
        nrm = jnp.where(sel, ni, nrm)
    nrm = jnp.maximum(jnp.sqrt(nrm), 1e-12)
    kn = kb / nrm

    cens = [c_ref[i] for i in range(G)]
    cns = [c / jnp.maximum(
        jnp.sqrt(jnp.sum(c * c, axis=-1, keepdims=True)), 1e-12)
        for c in cens]
    cnD = _block_diag(cns)

    simsT = jax.lax.dot_general(
        cnD, kn, (((1,), (1,)), ((), ())),
        preferred_element_type=jnp.float32)
    sims3 = simsT.reshape(G, C, S)

    oh3, _ = _first_max_mask_ax1(sims3)
    oh3 = oh3.astype(jnp.float32)
    counts = jnp.sum(oh3, axis=2, keepdims=True)
    ohT = oh3.reshape(G * C, S)
    ksumD = jax.lax.dot_general(
        ohT, kb, (((1,), (0,)), ((), ())),
        preferred_element_type=jnp.float32)
    vsumD = jax.lax.dot_general(
        ohT, vb, (((1,), (0,)), ((), ())),
        preferred_element_type=jnp.float32)

    has = counts > 0.0
    den = jnp.maximum(counts, 1.0)
    kmeans = []
    vmeans = []
    for i in range(G):
        ks = ksumD[C * i:C * (i + 1), hd * i:hd * (i + 1)]
        vs = vsumD[C * i:C * (i + 1), hd * i:hd * (i + 1)]
        kmeans.append(jnp.where(has[i], ks / den[i], cens[i]))
        vmeans.append(jnp.where(has[i], vs / den[i], 0.0))

    kmD = _block_diag(kmeans)
    scoresT = jax.lax.dot_general(
        kmD, qb, (((1,), (1,)), ((), ())),
        preferred_element_type=jnp.float32) * scale

    aw3 = _topk_softmax3(scoresT.reshape(G, C, S))

    awT = aw3.reshape(G * C, S)
    vmD = _block_diag(vmeans)
    y = jax.lax.dot_general(
        awT, vmD, (((0,), (0,)), ((), ())),
        preferred_element_type=jnp.float32)
    partial = jnp.dot(y, wo_ref[...],
                      preferred_element_type=jnp.float32)

    for i in range(G):
        aw_ref[0, i] = aw3[i].T

    @pl.when(h == 0)
    def _():
        o_ref[0] = partial + bo_ref[...]

    @pl.when(h != 0)
    def _():
        o_ref[0] += partial


def _fused_attn(qkv, cen, woT, bo, B, S, scale, interpret=False):
    C_, hd = cen.shape[1], cen.shape[2]
    D = qkv.shape[2]
    return pl.pallas_call(
        functools.partial(_fused_kernel, scale, hd),
        grid=(B, H // G),
        in_specs=[
            pl.BlockSpec((1, S, G * hd), lambda b, h: (0, b, h)),
            pl.BlockSpec((1, S, G * hd), lambda b, h: (1, b, h)),
            pl.BlockSpec((1, S, G * hd), lambda b, h: (2, b, h)),
            pl.BlockSpec((G, C_, hd), lambda b, h: (h, 0, 0)),
            pl.BlockSpec((G * hd, D), lambda b, h: (h, 0)),
            pl.BlockSpec((1, D), lambda b, h: (0, 0)),
        ],
        out_specs=[
            pl.BlockSpec((1, G, S, C_), lambda b, h: (b, h, 0, 0)),
            pl.BlockSpec((1, S, D), lambda b, h: (b, 0, 0)),
        ],
        out_shape=[
            jax.ShapeDtypeStruct((B, H, S, C_), jnp.float32),
            jax.ShapeDtypeStruct((B, S, D), jnp.float32),
        ],
        compiler_params=pltpu.CompilerParams(
            dimension_semantics=("parallel", "arbitrary")),
        interpret=interpret,
    )(qkv, qkv, qkv, cen, woT, bo.reshape(1, D))


def _impl(query, key, value, Wq, bq, Wk, bk, Wv, bv, Wo, bo, centroids,
          interpret=False):
    B, Qlen, D = query.shape
    S = key.shape[1]
    hd = D // H
    scale = hd ** (-0.5)

    qkv = _qkv_proj(query.reshape(B * Qlen, D), key.reshape(B * S, D),
                    value.reshape(B * S, D), Wq.T, Wk.T, Wv.T, bq, bk, bv,
                    bm=512, interpret=interpret)
    attn, out = _fused_attn(qkv, centroids, Wo.T, bo, B, S, scale,
                            interpret=interpret)
    return out, attn


def kernel(query, key, value, Wq, bq, Wk, bk, Wv, bv, Wo, bo, centroids):
    return _impl(query, key, value, Wq, bq, Wk, bk, Wv, bv, Wo, bo,
                 centroids)
